# Initial kernel scaffold; baseline (speedup 1.0000x reference)
import jax
import jax.numpy as jnp
from jax.experimental import pallas as pl


def kernel(x, edge_index, W1, a_src1, a_dst1, b1, W2, a_src2, a_dst2, b2, lin1_W, lin1_b, lin2_W, lin2_b, lin11_W, lin11_b, linV_W, linV_b):
    raise NotImplementedError("write your pallas kernel here")


# speedup vs baseline: 24.6713x; 24.6713x over previous
import functools

import jax
import jax.numpy as jnp
from jax import lax
from jax.experimental import pallas as pl
from jax.experimental.pallas import tpu as pltpu
from jax.experimental.pallas import tpu_sc as plsc

N = 10000
E = 320000
NC = 2
NS = 16
NW = NC * NS
EPT = E // NW
CH = 80
NCHUNK = EPT // CH
ROWBLK = 10
BR = N // ROWBLK


EPT2 = E // NS
NCHUNK2 = EPT2 // CH


def _make_sc_gat(D):
  D2 = D // 2
  mesh = plsc.VectorSubcoreMesh(core_axis_name="c", subcore_axis_name="s")
  rows_per_tile = N // NS

  @functools.partial(
      pl.kernel,
      out_type=(
          jax.ShapeDtypeStruct((NC, N, D2), jnp.float32),
          jax.ShapeDtypeStruct((NS, N), jnp.float32),
      ),
      mesh=mesh,
      compiler_params=pltpu.CompilerParams(use_tc_tiling_on_sc=False,
                                           needs_layout_passes=False),
      scratch_types=[
          pltpu.VMEM((N,), jnp.float32),
          pltpu.VMEM((N,), jnp.float32),
          pltpu.VMEM((NCHUNK2, CH), jnp.int32),
          pltpu.VMEM((NCHUNK2, CH), jnp.int32),
          pltpu.VMEM((N,), jnp.float32),
          pltpu.VMEM((CH, D2), jnp.float32),
          pltpu.VMEM_SHARED((N, D2), jnp.float32),
          pltpu.SemaphoreType.DMA,
          pltpu.SemaphoreType.DMA,
      ],
  )
  def sc_gat(hlo_hbm, hhi_hbm, asv_hbm, adv_hbm, src_hbm, dst_hbm,
             numer_hbm, denom_hbm,
             as_l, ad_l, src_l, dst_l, den_l, rows, numer_sp,
             gsem, ssem):
    cid = lax.axis_index("c")
    sid = lax.axis_index("s")

    pltpu.sync_copy(asv_hbm, as_l)
    pltpu.sync_copy(adv_hbm, ad_l)
    pltpu.sync_copy(src_hbm.at[sid], src_l)
    pltpu.sync_copy(dst_hbm.at[sid], dst_l)

    zero16 = jnp.zeros((16,), jnp.float32)

    def zden(i, carry):
      den_l[pl.ds(i * 16, 16)] = zero16
      return carry
    lax.fori_loop(0, N // 16, zden, 0)

    def zrowbuf(i, carry):
      rows[i, pl.ds(0, 16)] = zero16
      return carry
    nvec = CH * D2 // 16

    def zrowflat(i, carry):
      r = i // (D2 // 16)
      j = i % (D2 // 16)
      rows[r, pl.ds(j * 16, 16)] = zero16
      return carry
    del zrowbuf
    lax.fori_loop(0, nvec, zrowflat, 0)

    base = sid * rows_per_tile
    for t in range(rows_per_tile // CH):
      pltpu.sync_copy(rows, numer_sp.at[pl.ds(base + t * CH, CH)])
    rem = rows_per_tile - (rows_per_tile // CH) * CH
    pltpu.sync_copy(rows.at[pl.ds(0, rem)],
                    numer_sp.at[pl.ds(base + (rows_per_tile // CH) * CH, rem)])

    plsc.subcore_barrier()

    lane = lax.iota(jnp.int32, 16)

    def body(g, carry):
      @pl.when(cid == 0)
      def _():
        pltpu.async_copy(hlo_hbm.at[src_l.at[g]], rows, gsem).wait()

      @pl.when(cid == 1)
      def _():
        pltpu.async_copy(hhi_hbm.at[src_l.at[g]], rows, gsem).wait()

      for k in range(CH // 16):
        srcv = src_l[g, pl.ds(k * 16, 16)]
        dstv = dst_l[g, pl.ds(k * 16, 16)]
        av = plsc.load_gather(as_l, [srcv])
        bv = plsc.load_gather(ad_l, [dstv])
        e = av + bv
        e = jnp.where(e >= 0.0, e, 0.2 * e)
        s = jnp.exp(e)
        plsc.addupdate_scatter(den_l, [dstv], s)
        for i in range(16):
          si = jnp.full((16,), jnp.sum(jnp.where(lane == i, s, 0.0)))
          r = k * 16 + i
          for j in range(D2 // 16):
            sl = pl.ds(j * 16, 16)
            rows[r, sl] = rows[r, sl] * si
      pltpu.async_copy(rows, numer_sp.at[dst_l.at[g]], ssem, add=True).wait()
      return carry
    lax.fori_loop(0, NCHUNK2, body, 0)

    plsc.subcore_barrier()

    @pl.when(cid == 0)
    def _():
      pltpu.sync_copy(den_l, denom_hbm.at[sid])

    pltpu.sync_copy(numer_sp.at[pl.ds(base, rows_per_tile)],
                    numer_hbm.at[cid, pl.ds(base, rows_per_tile)])

  return sc_gat


_sc_gat64 = _make_sc_gat(64)
_sc_gat128 = _make_sc_gat(128)


def _proj_kernel(x_ref, w_ref, asrc_ref, adst_ref, h_ref, as_ref, ad_ref):
  h = jnp.dot(x_ref[...], w_ref[...], preferred_element_type=jnp.float32)
  h_ref[...] = h
  as_ref[...] = jnp.dot(h, asrc_ref[...], preferred_element_type=jnp.float32)
  ad_ref[...] = jnp.dot(h, adst_ref[...], preferred_element_type=jnp.float32)


def _proj(x, W, a_src, a_dst):
  din, dout = W.shape
  h, asv, adv = pl.pallas_call(
      _proj_kernel,
      grid=(ROWBLK,),
      in_specs=[
          pl.BlockSpec((BR, din), lambda i: (i, 0)),
          pl.BlockSpec((din, dout), lambda i: (0, 0)),
          pl.BlockSpec((dout, 1), lambda i: (0, 0)),
          pl.BlockSpec((dout, 1), lambda i: (0, 0)),
      ],
      out_specs=[
          pl.BlockSpec((BR, dout), lambda i: (i, 0)),
          pl.BlockSpec((BR, 1), lambda i: (i, 0)),
          pl.BlockSpec((BR, 1), lambda i: (i, 0)),
      ],
      out_shape=[
          jax.ShapeDtypeStruct((N, dout), jnp.float32),
          jax.ShapeDtypeStruct((N, 1), jnp.float32),
          jax.ShapeDtypeStruct((N, 1), jnp.float32),
      ],
  )(x, W, a_src.reshape(dout, 1), a_dst.reshape(dout, 1))
  return h, asv.reshape(N), adv.reshape(N)


def _mid_kernel(num_ref, den_ref, b_ref, w_ref, asrc_ref, adst_ref,
                h2_ref, as_ref, ad_ref):
  num = num_ref[...]
  den = jnp.sum(den_ref[...], axis=1) + 1e-16
  agg = num / den[:, None]
  hid = jnp.maximum(agg + b_ref[...], 0.0)
  h2 = jnp.dot(hid, w_ref[...], preferred_element_type=jnp.float32)
  h2_ref[...] = h2
  as_ref[...] = jnp.dot(h2, asrc_ref[...], preferred_element_type=jnp.float32)
  ad_ref[...] = jnp.dot(h2, adst_ref[...], preferred_element_type=jnp.float32)


def _mid(num1, den1, b1, W2, a_src2, a_dst2):
  din, dout = W2.shape
  h2, asv, adv = pl.pallas_call(
      _mid_kernel,
      grid=(ROWBLK,),
      in_specs=[
          pl.BlockSpec((BR, din), lambda i: (i, 0)),
          pl.BlockSpec((BR, NS), lambda i: (i, 0)),
          pl.BlockSpec((1, din), lambda i: (0, 0)),
          pl.BlockSpec((din, dout), lambda i: (0, 0)),
          pl.BlockSpec((dout, 1), lambda i: (0, 0)),
          pl.BlockSpec((dout, 1), lambda i: (0, 0)),
      ],
      out_specs=[
          pl.BlockSpec((BR, dout), lambda i: (i, 0)),
          pl.BlockSpec((BR, 1), lambda i: (i, 0)),
          pl.BlockSpec((BR, 1), lambda i: (i, 0)),
      ],
      out_shape=[
          jax.ShapeDtypeStruct((N, dout), jnp.float32),
          jax.ShapeDtypeStruct((N, 1), jnp.float32),
          jax.ShapeDtypeStruct((N, 1), jnp.float32),
      ],
  )(num1, den1, b1.reshape(1, din), W2,
    a_src2.reshape(dout, 1), a_dst2.reshape(dout, 1))
  return h2, asv.reshape(N), adv.reshape(N)


def _post_kernel(num_ref, den_ref, b_ref, w1_ref, b1_ref, w2_ref, b2_ref,
                 w11_ref, b11_ref, wv_ref, out_ref, vsum_ref):
  num = num_ref[...]
  den = jnp.sum(den_ref[...], axis=1) + 1e-16
  agg = num / den[:, None]
  hgat = jnp.maximum(agg + b_ref[...], 0.0)
  h = jnp.dot(hgat, w1_ref[...], preferred_element_type=jnp.float32) + b1_ref[...]
  h = jnp.dot(h, w2_ref[...], preferred_element_type=jnp.float32) + b2_ref[...]
  out_ref[...] = jnp.tanh(
      jnp.dot(h, w11_ref[...], preferred_element_type=jnp.float32) + b11_ref[...])
  vpart = jnp.sum(jnp.dot(h, wv_ref[...], preferred_element_type=jnp.float32))

  @pl.when(pl.program_id(0) == 0)
  def _():
    vsum_ref[...] = jnp.zeros_like(vsum_ref)

  vsum_ref[...] += jnp.reshape(vpart, (1, 1))


def _post(num2, den2, b2, lin1_W, lin1_b, lin2_W, lin2_b, lin11_W, lin11_b,
          linV_W):
  out, vsum = pl.pallas_call(
      _post_kernel,
      grid=(ROWBLK,),
      in_specs=[
          pl.BlockSpec((BR, 128), lambda i: (i, 0)),
          pl.BlockSpec((BR, NS), lambda i: (i, 0)),
          pl.BlockSpec((1, 128), lambda i: (0, 0)),
          pl.BlockSpec((128, 64), lambda i: (0, 0)),
          pl.BlockSpec((1, 64), lambda i: (0, 0)),
          pl.BlockSpec((64, 64), lambda i: (0, 0)),
          pl.BlockSpec((1, 64), lambda i: (0, 0)),
          pl.BlockSpec((64, 64), lambda i: (0, 0)),
          pl.BlockSpec((1, 64), lambda i: (0, 0)),
          pl.BlockSpec((64, 1), lambda i: (0, 0)),
      ],
      out_specs=[
          pl.BlockSpec((BR, 64), lambda i: (i, 0)),
          pl.BlockSpec((1, 1), lambda i: (0, 0)),
      ],
      out_shape=[
          jax.ShapeDtypeStruct((N, 64), jnp.float32),
          jax.ShapeDtypeStruct((1, 1), jnp.float32),
      ],
  )(num2, den2, b2.reshape(1, 128), lin1_W, lin1_b.reshape(1, 64),
    lin2_W, lin2_b.reshape(1, 64), lin11_W, lin11_b.reshape(1, 64), linV_W)
  return out, vsum


def kernel(x, edge_index, W1, a_src1, a_dst1, b1, W2, a_src2, a_dst2, b2,
           lin1_W, lin1_b, lin2_W, lin2_b, lin11_W, lin11_b, linV_W, linV_b):
  src = edge_index[0].reshape(NS, NCHUNK2, CH)
  dst = edge_index[1].reshape(NS, NCHUNK2, CH)

  h1, as1, ad1 = _proj(x, W1, a_src1, a_dst1)
  num1, den1 = _sc_gat64(h1[:, :32], h1[:, 32:], as1, ad1, src, dst)
  num1f = jnp.concatenate([num1[0], num1[1]], axis=1)
  den1 = den1.T
  h2, as2, ad2 = _mid(num1f, den1, b1, W2, a_src2, a_dst2)
  num2, den2 = _sc_gat128(h2[:, :64], h2[:, 64:], as2, ad2, src, dst)
  num2f = jnp.concatenate([num2[0], num2[1]], axis=1)
  den2 = den2.T
  out, vsum = _post(num2f, den2, b2, lin1_W, lin1_b, lin2_W, lin2_b,
                    lin11_W, lin11_b, linV_W)
  value = vsum[0, 0] / jnp.float32(N) + linV_b[0]
  return out, value


# speedup vs baseline: 36.9807x; 1.4989x over previous
import functools

import jax
import jax.numpy as jnp
from jax import lax
from jax.experimental import pallas as pl
from jax.experimental.pallas import tpu as pltpu
from jax.experimental.pallas import tpu_sc as plsc

N = 10000
E = 320000
NC = 2
NS = 16
NW = NC * NS
EPT = E // NW
CH = 80
NCHUNK = EPT // CH
ROWBLK = 10
BR = N // ROWBLK


EPT2 = E // NS
NCHUNK2 = EPT2 // CH


def _make_sc_gat(D):
  D2 = D // 2
  mesh = plsc.VectorSubcoreMesh(core_axis_name="c", subcore_axis_name="s")
  rows_per_tile = N // NS

  @functools.partial(
      pl.kernel,
      out_type=(
          jax.ShapeDtypeStruct((NC, N, D2), jnp.float32),
          jax.ShapeDtypeStruct((NS, N), jnp.float32),
      ),
      mesh=mesh,
      compiler_params=pltpu.CompilerParams(use_tc_tiling_on_sc=False,
                                           needs_layout_passes=False),
      scratch_types=[
          pltpu.VMEM((N,), jnp.float32),
          pltpu.VMEM((N,), jnp.float32),
          pltpu.VMEM((NCHUNK2, CH), jnp.int32),
          pltpu.VMEM((NCHUNK2, CH), jnp.int32),
          pltpu.VMEM((N,), jnp.float32),
          pltpu.VMEM((CH, D2), jnp.float32),
          pltpu.VMEM((CH, D2), jnp.float32),
          pltpu.VMEM_SHARED((N, D2), jnp.float32),
          pltpu.SemaphoreType.DMA,
          pltpu.SemaphoreType.DMA,
          pltpu.SemaphoreType.DMA,
          pltpu.SemaphoreType.DMA,
      ],
  )
  def sc_gat(hlo_hbm, hhi_hbm, asv_hbm, adv_hbm, src_hbm, dst_hbm,
             numer_hbm, denom_hbm,
             as_l, ad_l, src_l, dst_l, den_l, rows, rows2, numer_sp,
             gsem, ssem, gsem2, ssem2):
    cid = lax.axis_index("c")
    sid = lax.axis_index("s")

    pltpu.sync_copy(asv_hbm, as_l)
    pltpu.sync_copy(adv_hbm, ad_l)
    pltpu.sync_copy(src_hbm.at[sid], src_l)
    pltpu.sync_copy(dst_hbm.at[sid], dst_l)

    zero16 = jnp.zeros((16,), jnp.float32)

    def zden(i, carry):
      den_l[pl.ds(i * 16, 16)] = zero16
      return carry
    lax.fori_loop(0, N // 16, zden, 0)

    def zrowbuf(i, carry):
      rows[i, pl.ds(0, 16)] = zero16
      return carry
    nvec = CH * D2 // 16

    def zrowflat(i, carry):
      r = i // (D2 // 16)
      j = i % (D2 // 16)
      rows[r, pl.ds(j * 16, 16)] = zero16
      return carry
    del zrowbuf
    lax.fori_loop(0, nvec, zrowflat, 0)

    base = sid * rows_per_tile
    for t in range(rows_per_tile // CH):
      pltpu.sync_copy(rows, numer_sp.at[pl.ds(base + t * CH, CH)])
    rem = rows_per_tile - (rows_per_tile // CH) * CH
    pltpu.sync_copy(rows.at[pl.ds(0, rem)],
                    numer_sp.at[pl.ds(base + (rows_per_tile // CH) * CH, rem)])

    plsc.subcore_barrier()

    lane = lax.iota(jnp.int32, 16)

    def start_gather(g, buf, sem):
      @pl.when(cid == 0)
      def _():
        pltpu.async_copy(hlo_hbm.at[src_l.at[g]], buf, sem)

      @pl.when(cid == 1)
      def _():
        pltpu.async_copy(hhi_hbm.at[src_l.at[g]], buf, sem)

    def wait_gather(buf, sem):
      pltpu.make_async_copy(hlo_hbm.at[pl.ds(0, CH)], buf, sem).wait()

    def start_scatter(g, buf, sem):
      pltpu.async_copy(buf, numer_sp.at[dst_l.at[g]], sem, add=True)

    def wait_scatter(buf, sem):
      pltpu.make_async_copy(buf, numer_sp.at[pl.ds(0, CH)], sem).wait()

    def process(g, buf):
      for k in range(CH // 16):
        srcv = src_l[g, pl.ds(k * 16, 16)]
        dstv = dst_l[g, pl.ds(k * 16, 16)]
        av = plsc.load_gather(as_l, [srcv])
        bv = plsc.load_gather(ad_l, [dstv])
        e = av + bv
        e = jnp.where(e >= 0.0, e, 0.2 * e)
        s = jnp.exp(e)
        plsc.addupdate_scatter(den_l, [dstv], s)
        for i in range(16):
          si = jnp.full((16,), jnp.sum(jnp.where(lane == i, s, 0.0)))
          r = k * 16 + i
          for j in range(D2 // 16):
            sl = pl.ds(j * 16, 16)
            buf[r, sl] = buf[r, sl] * si

    NP = NCHUNK2 // 2
    start_gather(0, rows, gsem)

    def body(g2, carry):
      e = g2 * 2

      @pl.when(g2 > 0)
      def _():
        wait_scatter(rows2, ssem2)

      start_gather(e + 1, rows2, gsem2)
      wait_gather(rows, gsem)
      process(e, rows)
      start_scatter(e, rows, ssem)
      wait_gather(rows2, gsem2)
      process(e + 1, rows2)
      wait_scatter(rows, ssem)

      @pl.when(g2 + 1 < NP)
      def _():
        start_gather(e + 2, rows, gsem)

      start_scatter(e + 1, rows2, ssem2)
      return carry
    lax.fori_loop(0, NP, body, 0)
    wait_scatter(rows2, ssem2)

    plsc.subcore_barrier()

    @pl.when(cid == 0)
    def _():
      pltpu.sync_copy(den_l, denom_hbm.at[sid])

    pltpu.sync_copy(numer_sp.at[pl.ds(base, rows_per_tile)],
                    numer_hbm.at[cid, pl.ds(base, rows_per_tile)])

  return sc_gat


_sc_gat64 = _make_sc_gat(64)
_sc_gat128 = _make_sc_gat(128)


def _proj_kernel(x_ref, w_ref, asrc_ref, adst_ref, h_ref, as_ref, ad_ref):
  h = jnp.dot(x_ref[...], w_ref[...], preferred_element_type=jnp.float32)
  h_ref[...] = h
  as_ref[...] = jnp.dot(h, asrc_ref[...], preferred_element_type=jnp.float32)
  ad_ref[...] = jnp.dot(h, adst_ref[...], preferred_element_type=jnp.float32)


def _proj(x, W, a_src, a_dst):
  din, dout = W.shape
  h, asv, adv = pl.pallas_call(
      _proj_kernel,
      grid=(ROWBLK,),
      in_specs=[
          pl.BlockSpec((BR, din), lambda i: (i, 0)),
          pl.BlockSpec((din, dout), lambda i: (0, 0)),
          pl.BlockSpec((dout, 1), lambda i: (0, 0)),
          pl.BlockSpec((dout, 1), lambda i: (0, 0)),
      ],
      out_specs=[
          pl.BlockSpec((BR, dout), lambda i: (i, 0)),
          pl.BlockSpec((BR, 1), lambda i: (i, 0)),
          pl.BlockSpec((BR, 1), lambda i: (i, 0)),
      ],
      out_shape=[
          jax.ShapeDtypeStruct((N, dout), jnp.float32),
          jax.ShapeDtypeStruct((N, 1), jnp.float32),
          jax.ShapeDtypeStruct((N, 1), jnp.float32),
      ],
  )(x, W, a_src.reshape(dout, 1), a_dst.reshape(dout, 1))
  return h, asv.reshape(N), adv.reshape(N)


def _mid_kernel(num_ref, den_ref, b_ref, w_ref, asrc_ref, adst_ref,
                h2_ref, as_ref, ad_ref):
  num = num_ref[...]
  den = jnp.sum(den_ref[...], axis=1) + 1e-16
  agg = num / den[:, None]
  hid = jnp.maximum(agg + b_ref[...], 0.0)
  h2 = jnp.dot(hid, w_ref[...], preferred_element_type=jnp.float32)
  h2_ref[...] = h2
  as_ref[...] = jnp.dot(h2, asrc_ref[...], preferred_element_type=jnp.float32)
  ad_ref[...] = jnp.dot(h2, adst_ref[...], preferred_element_type=jnp.float32)


def _mid(num1, den1, b1, W2, a_src2, a_dst2):
  din, dout = W2.shape
  h2, asv, adv = pl.pallas_call(
      _mid_kernel,
      grid=(ROWBLK,),
      in_specs=[
          pl.BlockSpec((BR, din), lambda i: (i, 0)),
          pl.BlockSpec((BR, NS), lambda i: (i, 0)),
          pl.BlockSpec((1, din), lambda i: (0, 0)),
          pl.BlockSpec((din, dout), lambda i: (0, 0)),
          pl.BlockSpec((dout, 1), lambda i: (0, 0)),
          pl.BlockSpec((dout, 1), lambda i: (0, 0)),
      ],
      out_specs=[
          pl.BlockSpec((BR, dout), lambda i: (i, 0)),
          pl.BlockSpec((BR, 1), lambda i: (i, 0)),
          pl.BlockSpec((BR, 1), lambda i: (i, 0)),
      ],
      out_shape=[
          jax.ShapeDtypeStruct((N, dout), jnp.float32),
          jax.ShapeDtypeStruct((N, 1), jnp.float32),
          jax.ShapeDtypeStruct((N, 1), jnp.float32),
      ],
  )(num1, den1, b1.reshape(1, din), W2,
    a_src2.reshape(dout, 1), a_dst2.reshape(dout, 1))
  return h2, asv.reshape(N), adv.reshape(N)


def _post_kernel(num_ref, den_ref, b_ref, w1_ref, b1_ref, w2_ref, b2_ref,
                 w11_ref, b11_ref, wv_ref, out_ref, vsum_ref):
  num = num_ref[...]
  den = jnp.sum(den_ref[...], axis=1) + 1e-16
  agg = num / den[:, None]
  hgat = jnp.maximum(agg + b_ref[...], 0.0)
  h = jnp.dot(hgat, w1_ref[...], preferred_element_type=jnp.float32) + b1_ref[...]
  h = jnp.dot(h, w2_ref[...], preferred_element_type=jnp.float32) + b2_ref[...]
  out_ref[...] = jnp.tanh(
      jnp.dot(h, w11_ref[...], preferred_element_type=jnp.float32) + b11_ref[...])
  vpart = jnp.sum(jnp.dot(h, wv_ref[...], preferred_element_type=jnp.float32))

  @pl.when(pl.program_id(0) == 0)
  def _():
    vsum_ref[...] = jnp.zeros_like(vsum_ref)

  vsum_ref[...] += jnp.reshape(vpart, (1, 1))


def _post(num2, den2, b2, lin1_W, lin1_b, lin2_W, lin2_b, lin11_W, lin11_b,
          linV_W):
  out, vsum = pl.pallas_call(
      _post_kernel,
      grid=(ROWBLK,),
      in_specs=[
          pl.BlockSpec((BR, 128), lambda i: (i, 0)),
          pl.BlockSpec((BR, NS), lambda i: (i, 0)),
          pl.BlockSpec((1, 128), lambda i: (0, 0)),
          pl.BlockSpec((128, 64), lambda i: (0, 0)),
          pl.BlockSpec((1, 64), lambda i: (0, 0)),
          pl.BlockSpec((64, 64), lambda i: (0, 0)),
          pl.BlockSpec((1, 64), lambda i: (0, 0)),
          pl.BlockSpec((64, 64), lambda i: (0, 0)),
          pl.BlockSpec((1, 64), lambda i: (0, 0)),
          pl.BlockSpec((64, 1), lambda i: (0, 0)),
      ],
      out_specs=[
          pl.BlockSpec((BR, 64), lambda i: (i, 0)),
          pl.BlockSpec((1, 1), lambda i: (0, 0)),
      ],
      out_shape=[
          jax.ShapeDtypeStruct((N, 64), jnp.float32),
          jax.ShapeDtypeStruct((1, 1), jnp.float32),
      ],
  )(num2, den2, b2.reshape(1, 128), lin1_W, lin1_b.reshape(1, 64),
    lin2_W, lin2_b.reshape(1, 64), lin11_W, lin11_b.reshape(1, 64), linV_W)
  return out, vsum


def kernel(x, edge_index, W1, a_src1, a_dst1, b1, W2, a_src2, a_dst2, b2,
           lin1_W, lin1_b, lin2_W, lin2_b, lin11_W, lin11_b, linV_W, linV_b):
  src = edge_index[0].reshape(NS, NCHUNK2, CH)
  dst = edge_index[1].reshape(NS, NCHUNK2, CH)

  h1, as1, ad1 = _proj(x, W1, a_src1, a_dst1)
  num1, den1 = _sc_gat64(h1[:, :32], h1[:, 32:], as1, ad1, src, dst)
  num1f = jnp.concatenate([num1[0], num1[1]], axis=1)
  den1 = den1.T
  h2, as2, ad2 = _mid(num1f, den1, b1, W2, a_src2, a_dst2)
  num2, den2 = _sc_gat128(h2[:, :64], h2[:, 64:], as2, ad2, src, dst)
  num2f = jnp.concatenate([num2[0], num2[1]], axis=1)
  den2 = den2.T
  out, vsum = _post(num2f, den2, b2, lin1_W, lin1_b, lin2_W, lin2_b,
                    lin11_W, lin11_b, linV_W)
  value = vsum[0, 0] / jnp.float32(N) + linV_b[0]
  return out, value


# speedup vs baseline: 39.8238x; 1.0769x over previous
import functools

import jax
import jax.numpy as jnp
from jax import lax
from jax.experimental import pallas as pl
from jax.experimental.pallas import tpu as pltpu
from jax.experimental.pallas import tpu_sc as plsc

N = 10000
E = 320000
NC = 2
NS = 16
NW = NC * NS
EPT = E // NW
CH = 80
NCHUNK = EPT // CH
ROWBLK = 10
BR = N // ROWBLK


EPT2 = E // NS
NCHUNK2 = EPT2 // CH


def _make_sc_gat(D):
  D2 = D // 2
  mesh = plsc.VectorSubcoreMesh(core_axis_name="c", subcore_axis_name="s")
  rows_per_tile = N // NS

  @functools.partial(
      pl.kernel,
      out_type=(
          jax.ShapeDtypeStruct((NC, N, D2), jnp.float32),
          jax.ShapeDtypeStruct((NS, N), jnp.float32),
      ),
      mesh=mesh,
      compiler_params=pltpu.CompilerParams(use_tc_tiling_on_sc=False,
                                           needs_layout_passes=False),
      scratch_types=[
          pltpu.VMEM((N,), jnp.float32),
          pltpu.VMEM((N,), jnp.float32),
          pltpu.VMEM((NCHUNK2, CH), jnp.int32),
          pltpu.VMEM((NCHUNK2, CH), jnp.int32),
          pltpu.VMEM((N,), jnp.float32),
          pltpu.VMEM((CH, D2), jnp.float32),
          pltpu.VMEM((CH, D2), jnp.float32),
          pltpu.VMEM_SHARED((N, D2), jnp.float32),
          pltpu.SemaphoreType.DMA,
          pltpu.SemaphoreType.DMA,
          pltpu.SemaphoreType.DMA,
          pltpu.SemaphoreType.DMA,
      ],
  )
  def sc_gat(hlo_hbm, hhi_hbm, asv_hbm, adv_hbm, src_hbm, dst_hbm,
             numer_hbm, denom_hbm,
             as_l, ad_l, src_l, dst_l, den_l, rows, rows2, numer_sp,
             gsem, ssem, gsem2, ssem2):
    cid = lax.axis_index("c")
    sid = lax.axis_index("s")

    pltpu.sync_copy(asv_hbm, as_l)
    pltpu.sync_copy(adv_hbm, ad_l)
    pltpu.sync_copy(src_hbm.at[sid], src_l)
    pltpu.sync_copy(dst_hbm.at[sid], dst_l)

    zero16 = jnp.zeros((16,), jnp.float32)

    def zden(i, carry):
      den_l[pl.ds(i * 16, 16)] = zero16
      return carry
    lax.fori_loop(0, N // 16, zden, 0)

    def zrowbuf(i, carry):
      rows[i, pl.ds(0, 16)] = zero16
      return carry
    nvec = CH * D2 // 16

    def zrowflat(i, carry):
      r = i // (D2 // 16)
      j = i % (D2 // 16)
      rows[r, pl.ds(j * 16, 16)] = zero16
      return carry
    del zrowbuf
    lax.fori_loop(0, nvec, zrowflat, 0)

    base = sid * rows_per_tile
    for t in range(rows_per_tile // CH):
      pltpu.sync_copy(rows, numer_sp.at[pl.ds(base + t * CH, CH)])
    rem = rows_per_tile - (rows_per_tile // CH) * CH
    pltpu.sync_copy(rows.at[pl.ds(0, rem)],
                    numer_sp.at[pl.ds(base + (rows_per_tile // CH) * CH, rem)])

    plsc.subcore_barrier()

    lane = lax.iota(jnp.int32, 16)

    def start_gather(g, buf, sem):
      @pl.when(cid == 0)
      def _():
        pltpu.async_copy(hlo_hbm.at[src_l.at[g]], buf, sem)

      @pl.when(cid == 1)
      def _():
        pltpu.async_copy(hhi_hbm.at[src_l.at[g]], buf, sem)

    def wait_gather(buf, sem):
      pltpu.make_async_copy(hlo_hbm.at[pl.ds(0, CH)], buf, sem).wait()

    def start_scatter(g, buf, sem):
      pltpu.async_copy(buf, numer_sp.at[dst_l.at[g]], sem, add=True)

    def wait_scatter(buf, sem):
      pltpu.make_async_copy(buf, numer_sp.at[pl.ds(0, CH)], sem).wait()

    def process(g, buf):
      for k in range(CH // 16):
        srcv = src_l[g, pl.ds(k * 16, 16)]
        dstv = dst_l[g, pl.ds(k * 16, 16)]
        av = plsc.load_gather(as_l, [srcv])
        bv = plsc.load_gather(ad_l, [dstv])
        e = av + bv
        e = jnp.where(e >= 0.0, e, 0.2 * e)
        s = jnp.exp(e)
        plsc.addupdate_scatter(den_l, [dstv], s)
        for i in range(16):
          si = jnp.full((16,), s[i])
          r = k * 16 + i
          for j in range(D2 // 16):
            sl = pl.ds(j * 16, 16)
            buf[r, sl] = buf[r, sl] * si

    NP = NCHUNK2 // 2
    start_gather(0, rows, gsem)

    def body(g2, carry):
      e = g2 * 2

      @pl.when(g2 > 0)
      def _():
        wait_scatter(rows2, ssem2)

      start_gather(e + 1, rows2, gsem2)
      wait_gather(rows, gsem)
      process(e, rows)
      start_scatter(e, rows, ssem)
      wait_gather(rows2, gsem2)
      process(e + 1, rows2)
      wait_scatter(rows, ssem)

      @pl.when(g2 + 1 < NP)
      def _():
        start_gather(e + 2, rows, gsem)

      start_scatter(e + 1, rows2, ssem2)
      return carry
    lax.fori_loop(0, NP, body, 0)
    wait_scatter(rows2, ssem2)

    plsc.subcore_barrier()

    @pl.when(cid == 0)
    def _():
      pltpu.sync_copy(den_l, denom_hbm.at[sid])

    pltpu.sync_copy(numer_sp.at[pl.ds(base, rows_per_tile)],
                    numer_hbm.at[cid, pl.ds(base, rows_per_tile)])

  return sc_gat


_sc_gat64 = _make_sc_gat(64)
_sc_gat128 = _make_sc_gat(128)


def _proj_kernel(x_ref, w_ref, asrc_ref, adst_ref,
                 hlo_ref, hhi_ref, as_ref, ad_ref):
  h = jnp.dot(x_ref[...], w_ref[...], preferred_element_type=jnp.float32)
  d2 = h.shape[1] // 2
  hlo_ref[...] = h[:, :d2]
  hhi_ref[...] = h[:, d2:]
  as_ref[...] = jnp.dot(h, asrc_ref[...], preferred_element_type=jnp.float32)
  ad_ref[...] = jnp.dot(h, adst_ref[...], preferred_element_type=jnp.float32)


def _proj(x, W, a_src, a_dst):
  din, dout = W.shape
  d2 = dout // 2
  hlo, hhi, asv, adv = pl.pallas_call(
      _proj_kernel,
      grid=(ROWBLK,),
      in_specs=[
          pl.BlockSpec((BR, din), lambda i: (i, 0)),
          pl.BlockSpec((din, dout), lambda i: (0, 0)),
          pl.BlockSpec((dout, 1), lambda i: (0, 0)),
          pl.BlockSpec((dout, 1), lambda i: (0, 0)),
      ],
      out_specs=[
          pl.BlockSpec((BR, d2), lambda i: (i, 0)),
          pl.BlockSpec((BR, d2), lambda i: (i, 0)),
          pl.BlockSpec((BR, 1), lambda i: (i, 0)),
          pl.BlockSpec((BR, 1), lambda i: (i, 0)),
      ],
      out_shape=[
          jax.ShapeDtypeStruct((N, d2), jnp.float32),
          jax.ShapeDtypeStruct((N, d2), jnp.float32),
          jax.ShapeDtypeStruct((N, 1), jnp.float32),
          jax.ShapeDtypeStruct((N, 1), jnp.float32),
      ],
  )(x, W, a_src.reshape(dout, 1), a_dst.reshape(dout, 1))
  return hlo, hhi, asv.reshape(N), adv.reshape(N)


def _mid_kernel(num_ref, den_ref, b_ref, w_ref, asrc_ref, adst_ref,
                h2lo_ref, h2hi_ref, as_ref, ad_ref):
  den = jnp.sum(den_ref[...], axis=1) + 1e-16
  inv = 1.0 / den[:, None]
  d2 = num_ref.shape[2]
  hid_lo = jnp.maximum(num_ref[0] * inv + b_ref[:, :d2], 0.0)
  hid_hi = jnp.maximum(num_ref[1] * inv + b_ref[:, d2:], 0.0)
  h2 = (jnp.dot(hid_lo, w_ref[:d2], preferred_element_type=jnp.float32) +
        jnp.dot(hid_hi, w_ref[d2:], preferred_element_type=jnp.float32))
  do2 = h2.shape[1] // 2
  h2lo_ref[...] = h2[:, :do2]
  h2hi_ref[...] = h2[:, do2:]
  as_ref[...] = jnp.dot(h2, asrc_ref[...], preferred_element_type=jnp.float32)
  ad_ref[...] = jnp.dot(h2, adst_ref[...], preferred_element_type=jnp.float32)


def _mid(num1, den1t, b1, W2, a_src2, a_dst2):
  din, dout = W2.shape
  d2i = din // 2
  do2 = dout // 2
  h2lo, h2hi, asv, adv = pl.pallas_call(
      _mid_kernel,
      grid=(ROWBLK,),
      in_specs=[
          pl.BlockSpec((NC, BR, d2i), lambda i: (0, i, 0)),
          pl.BlockSpec((BR, NS), lambda i: (i, 0)),
          pl.BlockSpec((1, din), lambda i: (0, 0)),
          pl.BlockSpec((din, dout), lambda i: (0, 0)),
          pl.BlockSpec((dout, 1), lambda i: (0, 0)),
          pl.BlockSpec((dout, 1), lambda i: (0, 0)),
      ],
      out_specs=[
          pl.BlockSpec((BR, do2), lambda i: (i, 0)),
          pl.BlockSpec((BR, do2), lambda i: (i, 0)),
          pl.BlockSpec((BR, 1), lambda i: (i, 0)),
          pl.BlockSpec((BR, 1), lambda i: (i, 0)),
      ],
      out_shape=[
          jax.ShapeDtypeStruct((N, do2), jnp.float32),
          jax.ShapeDtypeStruct((N, do2), jnp.float32),
          jax.ShapeDtypeStruct((N, 1), jnp.float32),
          jax.ShapeDtypeStruct((N, 1), jnp.float32),
      ],
  )(num1, den1t, b1.reshape(1, din), W2,
    a_src2.reshape(dout, 1), a_dst2.reshape(dout, 1))
  return h2lo, h2hi, asv.reshape(N), adv.reshape(N)


def _post_kernel(num_ref, den_ref, b_ref, w1_ref, b1_ref, w2_ref, b2_ref,
                 w11_ref, b11_ref, wv_ref, out_ref, vsum_ref):
  den = jnp.sum(den_ref[...], axis=1) + 1e-16
  inv = 1.0 / den[:, None]
  d2 = num_ref.shape[2]
  hg_lo = jnp.maximum(num_ref[0] * inv + b_ref[:, :d2], 0.0)
  hg_hi = jnp.maximum(num_ref[1] * inv + b_ref[:, d2:], 0.0)
  h = (jnp.dot(hg_lo, w1_ref[:d2], preferred_element_type=jnp.float32) +
       jnp.dot(hg_hi, w1_ref[d2:], preferred_element_type=jnp.float32) +
       b1_ref[...])
  h = jnp.dot(h, w2_ref[...], preferred_element_type=jnp.float32) + b2_ref[...]
  out_ref[...] = jnp.tanh(
      jnp.dot(h, w11_ref[...], preferred_element_type=jnp.float32) + b11_ref[...])
  vpart = jnp.sum(jnp.dot(h, wv_ref[...], preferred_element_type=jnp.float32))

  @pl.when(pl.program_id(0) == 0)
  def _():
    vsum_ref[...] = jnp.zeros_like(vsum_ref)

  vsum_ref[...] += jnp.reshape(vpart, (1, 1))


def _post(num2, den2, b2, lin1_W, lin1_b, lin2_W, lin2_b, lin11_W, lin11_b,
          linV_W):
  out, vsum = pl.pallas_call(
      _post_kernel,
      grid=(ROWBLK,),
      in_specs=[
          pl.BlockSpec((NC, BR, 64), lambda i: (0, i, 0)),
          pl.BlockSpec((BR, NS), lambda i: (i, 0)),
          pl.BlockSpec((1, 128), lambda i: (0, 0)),
          pl.BlockSpec((128, 64), lambda i: (0, 0)),
          pl.BlockSpec((1, 64), lambda i: (0, 0)),
          pl.BlockSpec((64, 64), lambda i: (0, 0)),
          pl.BlockSpec((1, 64), lambda i: (0, 0)),
          pl.BlockSpec((64, 64), lambda i: (0, 0)),
          pl.BlockSpec((1, 64), lambda i: (0, 0)),
          pl.BlockSpec((64, 1), lambda i: (0, 0)),
      ],
      out_specs=[
          pl.BlockSpec((BR, 64), lambda i: (i, 0)),
          pl.BlockSpec((1, 1), lambda i: (0, 0)),
      ],
      out_shape=[
          jax.ShapeDtypeStruct((N, 64), jnp.float32),
          jax.ShapeDtypeStruct((1, 1), jnp.float32),
      ],
  )(num2, den2, b2.reshape(1, 128), lin1_W, lin1_b.reshape(1, 64),
    lin2_W, lin2_b.reshape(1, 64), lin11_W, lin11_b.reshape(1, 64), linV_W)
  return out, vsum


def kernel(x, edge_index, W1, a_src1, a_dst1, b1, W2, a_src2, a_dst2, b2,
           lin1_W, lin1_b, lin2_W, lin2_b, lin11_W, lin11_b, linV_W, linV_b):
  src = edge_index[0].reshape(NS, NCHUNK2, CH)
  dst = edge_index[1].reshape(NS, NCHUNK2, CH)

  h1lo, h1hi, as1, ad1 = _proj(x, W1, a_src1, a_dst1)
  num1, den1 = _sc_gat64(h1lo, h1hi, as1, ad1, src, dst)
  h2lo, h2hi, as2, ad2 = _mid(num1, den1.T, b1, W2, a_src2, a_dst2)
  num2, den2 = _sc_gat128(h2lo, h2hi, as2, ad2, src, dst)
  out, vsum = _post(num2, den2.T, b2, lin1_W, lin1_b, lin2_W, lin2_b,
                    lin11_W, lin11_b, linV_W)
  value = vsum[0, 0] / jnp.float32(N) + linV_b[0]
  return out, value


# speedup vs baseline: 41.2439x; 1.0357x over previous
import functools

import jax
import jax.numpy as jnp
from jax import lax
from jax.experimental import pallas as pl
from jax.experimental.pallas import tpu as pltpu
from jax.experimental.pallas import tpu_sc as plsc

N = 10000
E = 320000
NC = 2
NS = 16
NW = NC * NS
EPT = E // NW
CH = 80
NCHUNK = EPT // CH
ROWBLK = 10
BR = N // ROWBLK


EPT2 = E // NS
NCHUNK2 = EPT2 // CH


def _make_sc_gat(D):
  D2 = D // 2
  mesh = plsc.VectorSubcoreMesh(core_axis_name="c", subcore_axis_name="s")
  rows_per_tile = N // NS

  @functools.partial(
      pl.kernel,
      out_type=(
          jax.ShapeDtypeStruct((NC, N, D2), jnp.float32),
          jax.ShapeDtypeStruct((NS, N), jnp.float32),
      ),
      mesh=mesh,
      compiler_params=pltpu.CompilerParams(use_tc_tiling_on_sc=False,
                                           needs_layout_passes=False),
      scratch_types=[
          pltpu.VMEM((N,), jnp.float32),
          pltpu.VMEM((N,), jnp.float32),
          pltpu.VMEM((NCHUNK2, CH), jnp.int32),
          pltpu.VMEM((NCHUNK2, CH), jnp.int32),
          pltpu.VMEM((N,), jnp.float32),
          pltpu.VMEM((CH, D2), jnp.float32),
          pltpu.VMEM((CH, D2), jnp.float32),
          pltpu.VMEM((CH, D2), jnp.float32),
          pltpu.VMEM((CH, D2), jnp.float32),
          pltpu.VMEM_SHARED((N, D2), jnp.float32),
          pltpu.SemaphoreType.DMA,
          pltpu.SemaphoreType.DMA,
          pltpu.SemaphoreType.DMA,
          pltpu.SemaphoreType.DMA,
          pltpu.SemaphoreType.DMA,
          pltpu.SemaphoreType.DMA,
          pltpu.SemaphoreType.DMA,
          pltpu.SemaphoreType.DMA,
      ],
  )
  def sc_gat(hlo_hbm, hhi_hbm, asv_hbm, adv_hbm, src_hbm, dst_hbm,
             numer_hbm, denom_hbm,
             as_l, ad_l, src_l, dst_l, den_l, rows, rows2, rows3, rows4,
             numer_sp, gsem, ssem, gsem2, ssem2, gsem3, ssem3, gsem4, ssem4):
    cid = lax.axis_index("c")
    sid = lax.axis_index("s")

    pltpu.sync_copy(asv_hbm, as_l)
    pltpu.sync_copy(adv_hbm, ad_l)
    pltpu.sync_copy(src_hbm.at[sid], src_l)
    pltpu.sync_copy(dst_hbm.at[sid], dst_l)

    zero16 = jnp.zeros((16,), jnp.float32)

    def zden(i, carry):
      den_l[pl.ds(i * 16, 16)] = zero16
      return carry
    lax.fori_loop(0, N // 16, zden, 0)

    def zrowbuf(i, carry):
      rows[i, pl.ds(0, 16)] = zero16
      return carry
    nvec = CH * D2 // 16

    def zrowflat(i, carry):
      r = i // (D2 // 16)
      j = i % (D2 // 16)
      rows[r, pl.ds(j * 16, 16)] = zero16
      return carry
    del zrowbuf
    lax.fori_loop(0, nvec, zrowflat, 0)

    base = sid * rows_per_tile
    for t in range(rows_per_tile // CH):
      pltpu.sync_copy(rows, numer_sp.at[pl.ds(base + t * CH, CH)])
    rem = rows_per_tile - (rows_per_tile // CH) * CH
    pltpu.sync_copy(rows.at[pl.ds(0, rem)],
                    numer_sp.at[pl.ds(base + (rows_per_tile // CH) * CH, rem)])

    plsc.subcore_barrier()

    lane = lax.iota(jnp.int32, 16)

    def start_gather(g, buf, sem):
      @pl.when(cid == 0)
      def _():
        pltpu.async_copy(hlo_hbm.at[src_l.at[g]], buf, sem)

      @pl.when(cid == 1)
      def _():
        pltpu.async_copy(hhi_hbm.at[src_l.at[g]], buf, sem)

    def wait_gather(buf, sem):
      pltpu.make_async_copy(hlo_hbm.at[pl.ds(0, CH)], buf, sem).wait()

    def start_scatter(g, buf, sem):
      pltpu.async_copy(buf, numer_sp.at[dst_l.at[g]], sem, add=True)

    def wait_scatter(buf, sem):
      pltpu.make_async_copy(buf, numer_sp.at[pl.ds(0, CH)], sem).wait()

    def process(g, buf):
      for k in range(CH // 16):
        srcv = src_l[g, pl.ds(k * 16, 16)]
        dstv = dst_l[g, pl.ds(k * 16, 16)]
        av = plsc.load_gather(as_l, [srcv])
        bv = plsc.load_gather(ad_l, [dstv])
        e = av + bv
        e = jnp.where(e >= 0.0, e, 0.2 * e)
        s = jnp.exp(e)
        plsc.addupdate_scatter(den_l, [dstv], s)
        for i in range(16):
          si = jnp.full((16,), s[i])
          r = k * 16 + i
          for j in range(D2 // 16):
            sl = pl.ds(j * 16, 16)
            buf[r, sl] = buf[r, sl] * si

    bufs = (rows, rows2, rows3, rows4)
    gsems = (gsem, gsem2, gsem3, gsem4)
    ssems = (ssem, ssem2, ssem3, ssem4)
    NB = NCHUNK2 // 4
    start_gather(0, bufs[0], gsems[0])
    start_gather(1, bufs[1], gsems[1])

    def step(c, q):
      b = bufs[q]
      wait_gather(b, gsems[q])
      process(c, b)
      start_scatter(c, b, ssems[q])
      qn = (q + 2) % 4

      @pl.when(c + 2 < NCHUNK2)
      def _():
        @pl.when(c + 2 >= 4)
        def _():
          wait_scatter(bufs[qn], ssems[qn])
        start_gather(c + 2, bufs[qn], gsems[qn])

    def body(g4, carry):
      c0 = g4 * 4
      for q in range(4):
        step(c0 + q, q)
      return carry
    lax.fori_loop(0, NB, body, 0)
    for q in range(NCHUNK2 - NB * 4):
      step(NB * 4 + q, q)
    wait_scatter(bufs[0], ssems[0])
    wait_scatter(bufs[1], ssems[1])
    wait_scatter(bufs[2], ssems[2])
    wait_scatter(bufs[3], ssems[3])

    plsc.subcore_barrier()

    @pl.when(cid == 0)
    def _():
      pltpu.sync_copy(den_l, denom_hbm.at[sid])

    pltpu.sync_copy(numer_sp.at[pl.ds(base, rows_per_tile)],
                    numer_hbm.at[cid, pl.ds(base, rows_per_tile)])

  return sc_gat


_sc_gat64 = _make_sc_gat(64)
_sc_gat128 = _make_sc_gat(128)


def _proj_kernel(x_ref, w_ref, asrc_ref, adst_ref,
                 hlo_ref, hhi_ref, as_ref, ad_ref):
  h = jnp.dot(x_ref[...], w_ref[...], preferred_element_type=jnp.float32)
  d2 = h.shape[1] // 2
  hlo_ref[...] = h[:, :d2]
  hhi_ref[...] = h[:, d2:]
  as_ref[...] = jnp.dot(h, asrc_ref[...], preferred_element_type=jnp.float32)
  ad_ref[...] = jnp.dot(h, adst_ref[...], preferred_element_type=jnp.float32)


def _proj(x, W, a_src, a_dst):
  din, dout = W.shape
  d2 = dout // 2
  hlo, hhi, asv, adv = pl.pallas_call(
      _proj_kernel,
      grid=(ROWBLK,),
      in_specs=[
          pl.BlockSpec((BR, din), lambda i: (i, 0)),
          pl.BlockSpec((din, dout), lambda i: (0, 0)),
          pl.BlockSpec((dout, 1), lambda i: (0, 0)),
          pl.BlockSpec((dout, 1), lambda i: (0, 0)),
      ],
      out_specs=[
          pl.BlockSpec((BR, d2), lambda i: (i, 0)),
          pl.BlockSpec((BR, d2), lambda i: (i, 0)),
          pl.BlockSpec((BR, 1), lambda i: (i, 0)),
          pl.BlockSpec((BR, 1), lambda i: (i, 0)),
      ],
      out_shape=[
          jax.ShapeDtypeStruct((N, d2), jnp.float32),
          jax.ShapeDtypeStruct((N, d2), jnp.float32),
          jax.ShapeDtypeStruct((N, 1), jnp.float32),
          jax.ShapeDtypeStruct((N, 1), jnp.float32),
      ],
  )(x, W, a_src.reshape(dout, 1), a_dst.reshape(dout, 1))
  return hlo, hhi, asv.reshape(N), adv.reshape(N)


def _mid_kernel(num_ref, den_ref, b_ref, w_ref, asrc_ref, adst_ref,
                h2lo_ref, h2hi_ref, as_ref, ad_ref):
  den = jnp.sum(den_ref[...], axis=1) + 1e-16
  inv = 1.0 / den[:, None]
  d2 = num_ref.shape[2]
  hid_lo = jnp.maximum(num_ref[0] * inv + b_ref[:, :d2], 0.0)
  hid_hi = jnp.maximum(num_ref[1] * inv + b_ref[:, d2:], 0.0)
  h2 = (jnp.dot(hid_lo, w_ref[:d2], preferred_element_type=jnp.float32) +
        jnp.dot(hid_hi, w_ref[d2:], preferred_element_type=jnp.float32))
  do2 = h2.shape[1] // 2
  h2lo_ref[...] = h2[:, :do2]
  h2hi_ref[...] = h2[:, do2:]
  as_ref[...] = jnp.dot(h2, asrc_ref[...], preferred_element_type=jnp.float32)
  ad_ref[...] = jnp.dot(h2, adst_ref[...], preferred_element_type=jnp.float32)


def _mid(num1, den1t, b1, W2, a_src2, a_dst2):
  din, dout = W2.shape
  d2i = din // 2
  do2 = dout // 2
  h2lo, h2hi, asv, adv = pl.pallas_call(
      _mid_kernel,
      grid=(ROWBLK,),
      in_specs=[
          pl.BlockSpec((NC, BR, d2i), lambda i: (0, i, 0)),
          pl.BlockSpec((BR, NS), lambda i: (i, 0)),
          pl.BlockSpec((1, din), lambda i: (0, 0)),
          pl.BlockSpec((din, dout), lambda i: (0, 0)),
          pl.BlockSpec((dout, 1), lambda i: (0, 0)),
          pl.BlockSpec((dout, 1), lambda i: (0, 0)),
      ],
      out_specs=[
          pl.BlockSpec((BR, do2), lambda i: (i, 0)),
          pl.BlockSpec((BR, do2), lambda i: (i, 0)),
          pl.BlockSpec((BR, 1), lambda i: (i, 0)),
          pl.BlockSpec((BR, 1), lambda i: (i, 0)),
      ],
      out_shape=[
          jax.ShapeDtypeStruct((N, do2), jnp.float32),
          jax.ShapeDtypeStruct((N, do2), jnp.float32),
          jax.ShapeDtypeStruct((N, 1), jnp.float32),
          jax.ShapeDtypeStruct((N, 1), jnp.float32),
      ],
  )(num1, den1t, b1.reshape(1, din), W2,
    a_src2.reshape(dout, 1), a_dst2.reshape(dout, 1))
  return h2lo, h2hi, asv.reshape(N), adv.reshape(N)


def _post_kernel(num_ref, den_ref, b_ref, w1_ref, b1_ref, w2_ref, b2_ref,
                 w11_ref, b11_ref, wv_ref, out_ref, vsum_ref):
  den = jnp.sum(den_ref[...], axis=1) + 1e-16
  inv = 1.0 / den[:, None]
  d2 = num_ref.shape[2]
  hg_lo = jnp.maximum(num_ref[0] * inv + b_ref[:, :d2], 0.0)
  hg_hi = jnp.maximum(num_ref[1] * inv + b_ref[:, d2:], 0.0)
  h = (jnp.dot(hg_lo, w1_ref[:d2], preferred_element_type=jnp.float32) +
       jnp.dot(hg_hi, w1_ref[d2:], preferred_element_type=jnp.float32) +
       b1_ref[...])
  h = jnp.dot(h, w2_ref[...], preferred_element_type=jnp.float32) + b2_ref[...]
  out_ref[...] = jnp.tanh(
      jnp.dot(h, w11_ref[...], preferred_element_type=jnp.float32) + b11_ref[...])
  vpart = jnp.sum(jnp.dot(h, wv_ref[...], preferred_element_type=jnp.float32))

  @pl.when(pl.program_id(0) == 0)
  def _():
    vsum_ref[...] = jnp.zeros_like(vsum_ref)

  vsum_ref[...] += jnp.reshape(vpart, (1, 1))


def _post(num2, den2, b2, lin1_W, lin1_b, lin2_W, lin2_b, lin11_W, lin11_b,
          linV_W):
  out, vsum = pl.pallas_call(
      _post_kernel,
      grid=(ROWBLK,),
      in_specs=[
          pl.BlockSpec((NC, BR, 64), lambda i: (0, i, 0)),
          pl.BlockSpec((BR, NS), lambda i: (i, 0)),
          pl.BlockSpec((1, 128), lambda i: (0, 0)),
          pl.BlockSpec((128, 64), lambda i: (0, 0)),
          pl.BlockSpec((1, 64), lambda i: (0, 0)),
          pl.BlockSpec((64, 64), lambda i: (0, 0)),
          pl.BlockSpec((1, 64), lambda i: (0, 0)),
          pl.BlockSpec((64, 64), lambda i: (0, 0)),
          pl.BlockSpec((1, 64), lambda i: (0, 0)),
          pl.BlockSpec((64, 1), lambda i: (0, 0)),
      ],
      out_specs=[
          pl.BlockSpec((BR, 64), lambda i: (i, 0)),
          pl.BlockSpec((1, 1), lambda i: (0, 0)),
      ],
      out_shape=[
          jax.ShapeDtypeStruct((N, 64), jnp.float32),
          jax.ShapeDtypeStruct((1, 1), jnp.float32),
      ],
  )(num2, den2, b2.reshape(1, 128), lin1_W, lin1_b.reshape(1, 64),
    lin2_W, lin2_b.reshape(1, 64), lin11_W, lin11_b.reshape(1, 64), linV_W)
  return out, vsum


def kernel(x, edge_index, W1, a_src1, a_dst1, b1, W2, a_src2, a_dst2, b2,
           lin1_W, lin1_b, lin2_W, lin2_b, lin11_W, lin11_b, linV_W, linV_b):
  src = edge_index[0].reshape(NS, NCHUNK2, CH)
  dst = edge_index[1].reshape(NS, NCHUNK2, CH)

  h1lo, h1hi, as1, ad1 = _proj(x, W1, a_src1, a_dst1)
  num1, den1 = _sc_gat64(h1lo, h1hi, as1, ad1, src, dst)
  h2lo, h2hi, as2, ad2 = _mid(num1, den1.T, b1, W2, a_src2, a_dst2)
  num2, den2 = _sc_gat128(h2lo, h2hi, as2, ad2, src, dst)
  out, vsum = _post(num2, den2.T, b2, lin1_W, lin1_b, lin2_W, lin2_b,
                    lin11_W, lin11_b, linV_W)
  value = vsum[0, 0] / jnp.float32(N) + linV_b[0]
  return out, value


# speedup vs baseline: 42.4942x; 1.0303x over previous
import functools

import jax
import jax.numpy as jnp
from jax import lax
from jax.experimental import pallas as pl
from jax.experimental.pallas import tpu as pltpu
from jax.experimental.pallas import tpu_sc as plsc

N = 10000
E = 320000
NC = 2
NS = 16
NW = NC * NS
EPT = E // NW
CH = 80
NCHUNK = EPT // CH
ROWBLK = 10
BR = N // ROWBLK


EPT2 = E // NS
NCHUNK2 = EPT2 // CH


def _make_sc_gat(D):
  D2 = D // 2
  mesh = plsc.VectorSubcoreMesh(core_axis_name="c", subcore_axis_name="s")
  rows_per_tile = N // NS

  @functools.partial(
      pl.kernel,
      out_type=(
          jax.ShapeDtypeStruct((NC, N, D2), jnp.float32),
          jax.ShapeDtypeStruct((NS, N), jnp.float32),
      ),
      mesh=mesh,
      compiler_params=pltpu.CompilerParams(use_tc_tiling_on_sc=False,
                                           needs_layout_passes=False),
      scratch_types=[
          pltpu.VMEM((N,), jnp.float32),
          pltpu.VMEM((N,), jnp.float32),
          pltpu.VMEM((NCHUNK2, CH), jnp.int32),
          pltpu.VMEM((NCHUNK2, CH), jnp.int32),
          pltpu.VMEM((N,), jnp.float32),
          pltpu.VMEM((CH, D2), jnp.float32),
          pltpu.VMEM((CH, D2), jnp.float32),
          pltpu.VMEM((CH, D2), jnp.float32),
          pltpu.VMEM((CH, D2), jnp.float32),
          pltpu.VMEM_SHARED((N, D2), jnp.float32),
          pltpu.SemaphoreType.DMA,
          pltpu.SemaphoreType.DMA,
          pltpu.SemaphoreType.DMA,
          pltpu.SemaphoreType.DMA,
          pltpu.SemaphoreType.DMA,
          pltpu.SemaphoreType.DMA,
          pltpu.SemaphoreType.DMA,
          pltpu.SemaphoreType.DMA,
      ],
  )
  def sc_gat(hlo_hbm, hhi_hbm, asv_hbm, adv_hbm, edge_hbm,
             numer_hbm, denom_hbm,
             as_l, ad_l, src_l, dst_l, den_l, rows, rows2, rows3, rows4,
             numer_sp, gsem, ssem, gsem2, ssem2, gsem3, ssem3, gsem4, ssem4):
    cid = lax.axis_index("c")
    sid = lax.axis_index("s")

    c1 = pltpu.async_copy(asv_hbm, as_l, gsem)
    c2 = pltpu.async_copy(adv_hbm, ad_l, ssem)
    c3 = pltpu.async_copy(edge_hbm.at[0, sid], src_l, gsem2)
    c4 = pltpu.async_copy(edge_hbm.at[1, sid], dst_l, ssem2)
    c1.wait()
    c2.wait()
    c3.wait()
    c4.wait()

    zero16 = jnp.zeros((16,), jnp.float32)

    def zden(i, carry):
      den_l[pl.ds(i * 16, 16)] = zero16
      return carry
    lax.fori_loop(0, N // 16, zden, 0)

    def zrowbuf(i, carry):
      rows[i, pl.ds(0, 16)] = zero16
      return carry
    nvec = CH * D2 // 16

    def zrowflat(i, carry):
      r = i // (D2 // 16)
      j = i % (D2 // 16)
      rows[r, pl.ds(j * 16, 16)] = zero16
      return carry
    del zrowbuf
    lax.fori_loop(0, nvec, zrowflat, 0)

    base = sid * rows_per_tile
    for t in range(rows_per_tile // CH):
      pltpu.sync_copy(rows, numer_sp.at[pl.ds(base + t * CH, CH)])
    rem = rows_per_tile - (rows_per_tile // CH) * CH
    pltpu.sync_copy(rows.at[pl.ds(0, rem)],
                    numer_sp.at[pl.ds(base + (rows_per_tile // CH) * CH, rem)])

    plsc.subcore_barrier()

    lane = lax.iota(jnp.int32, 16)

    def start_gather(g, buf, sem):
      @pl.when(cid == 0)
      def _():
        pltpu.async_copy(hlo_hbm.at[src_l.at[g]], buf, sem)

      @pl.when(cid == 1)
      def _():
        pltpu.async_copy(hhi_hbm.at[src_l.at[g]], buf, sem)

    def wait_gather(buf, sem):
      pltpu.make_async_copy(hlo_hbm.at[pl.ds(0, CH)], buf, sem).wait()

    def start_scatter(g, buf, sem):
      pltpu.async_copy(buf, numer_sp.at[dst_l.at[g]], sem, add=True)

    def wait_scatter(buf, sem):
      pltpu.make_async_copy(buf, numer_sp.at[pl.ds(0, CH)], sem).wait()

    def process(g, buf):
      for k in range(CH // 16):
        srcv = src_l[g, pl.ds(k * 16, 16)]
        dstv = dst_l[g, pl.ds(k * 16, 16)]
        av = plsc.load_gather(as_l, [srcv])
        bv = plsc.load_gather(ad_l, [dstv])
        e = av + bv
        e = jnp.where(e >= 0.0, e, 0.2 * e)
        s = jnp.exp(e)
        plsc.addupdate_scatter(den_l, [dstv], s)
        for i in range(16):
          si = jnp.full((16,), s[i])
          r = k * 16 + i
          for j in range(D2 // 16):
            sl = pl.ds(j * 16, 16)
            buf[r, sl] = buf[r, sl] * si

    bufs = (rows, rows2, rows3, rows4)
    gsems = (gsem, gsem2, gsem3, gsem4)
    ssems = (ssem, ssem2, ssem3, ssem4)
    NB = NCHUNK2 // 4
    start_gather(0, bufs[0], gsems[0])
    start_gather(1, bufs[1], gsems[1])

    def step(c, q):
      b = bufs[q]
      wait_gather(b, gsems[q])
      process(c, b)
      start_scatter(c, b, ssems[q])
      qn = (q + 2) % 4

      @pl.when(c + 2 < NCHUNK2)
      def _():
        @pl.when(c + 2 >= 4)
        def _():
          wait_scatter(bufs[qn], ssems[qn])
        start_gather(c + 2, bufs[qn], gsems[qn])

    def body(g4, carry):
      c0 = g4 * 4
      for q in range(4):
        step(c0 + q, q)
      return carry
    lax.fori_loop(0, NB, body, 0)
    for q in range(NCHUNK2 - NB * 4):
      step(NB * 4 + q, q)
    wait_scatter(bufs[0], ssems[0])
    wait_scatter(bufs[1], ssems[1])
    wait_scatter(bufs[2], ssems[2])
    wait_scatter(bufs[3], ssems[3])

    plsc.subcore_barrier()

    @pl.when(cid == 0)
    def _():
      pltpu.sync_copy(den_l, denom_hbm.at[sid])

    pltpu.sync_copy(numer_sp.at[pl.ds(base, rows_per_tile)],
                    numer_hbm.at[cid, pl.ds(base, rows_per_tile)])

  return sc_gat


_sc_gat64 = _make_sc_gat(64)
_sc_gat128 = _make_sc_gat(128)


def _proj_kernel(x_ref, w_ref, asrc_ref, adst_ref,
                 hlo_ref, hhi_ref, as_ref, ad_ref):
  h = jnp.dot(x_ref[...], w_ref[...], preferred_element_type=jnp.float32)
  d2 = h.shape[1] // 2
  hlo_ref[...] = h[:, :d2]
  hhi_ref[...] = h[:, d2:]
  as_ref[...] = jnp.dot(h, asrc_ref[...], preferred_element_type=jnp.float32)
  ad_ref[...] = jnp.dot(h, adst_ref[...], preferred_element_type=jnp.float32)


def _proj(x, W, a_src, a_dst):
  din, dout = W.shape
  d2 = dout // 2
  hlo, hhi, asv, adv = pl.pallas_call(
      _proj_kernel,
      grid=(ROWBLK,),
      in_specs=[
          pl.BlockSpec((BR, din), lambda i: (i, 0)),
          pl.BlockSpec((din, dout), lambda i: (0, 0)),
          pl.BlockSpec((dout, 1), lambda i: (0, 0)),
          pl.BlockSpec((dout, 1), lambda i: (0, 0)),
      ],
      out_specs=[
          pl.BlockSpec((BR, d2), lambda i: (i, 0)),
          pl.BlockSpec((BR, d2), lambda i: (i, 0)),
          pl.BlockSpec((BR, 1), lambda i: (i, 0)),
          pl.BlockSpec((BR, 1), lambda i: (i, 0)),
      ],
      out_shape=[
          jax.ShapeDtypeStruct((N, d2), jnp.float32),
          jax.ShapeDtypeStruct((N, d2), jnp.float32),
          jax.ShapeDtypeStruct((N, 1), jnp.float32),
          jax.ShapeDtypeStruct((N, 1), jnp.float32),
      ],
  )(x, W, a_src.reshape(dout, 1), a_dst.reshape(dout, 1))
  return hlo, hhi, asv, adv


def _mid_kernel(num_ref, den_ref, b_ref, w_ref, asrc_ref, adst_ref,
                h2lo_ref, h2hi_ref, as_ref, ad_ref):
  den = jnp.sum(den_ref[...], axis=1) + 1e-16
  inv = 1.0 / den[:, None]
  d2 = num_ref.shape[2]
  hid_lo = jnp.maximum(num_ref[0] * inv + b_ref[:, :d2], 0.0)
  hid_hi = jnp.maximum(num_ref[1] * inv + b_ref[:, d2:], 0.0)
  h2 = (jnp.dot(hid_lo, w_ref[:d2], preferred_element_type=jnp.float32) +
        jnp.dot(hid_hi, w_ref[d2:], preferred_element_type=jnp.float32))
  do2 = h2.shape[1] // 2
  h2lo_ref[...] = h2[:, :do2]
  h2hi_ref[...] = h2[:, do2:]
  as_ref[...] = jnp.dot(h2, asrc_ref[...], preferred_element_type=jnp.float32)
  ad_ref[...] = jnp.dot(h2, adst_ref[...], preferred_element_type=jnp.float32)


def _mid(num1, den1t, b1, W2, a_src2, a_dst2):
  din, dout = W2.shape
  d2i = din // 2
  do2 = dout // 2
  h2lo, h2hi, asv, adv = pl.pallas_call(
      _mid_kernel,
      grid=(ROWBLK,),
      in_specs=[
          pl.BlockSpec((NC, BR, d2i), lambda i: (0, i, 0)),
          pl.BlockSpec((BR, NS), lambda i: (i, 0)),
          pl.BlockSpec((1, din), lambda i: (0, 0)),
          pl.BlockSpec((din, dout), lambda i: (0, 0)),
          pl.BlockSpec((dout, 1), lambda i: (0, 0)),
          pl.BlockSpec((dout, 1), lambda i: (0, 0)),
      ],
      out_specs=[
          pl.BlockSpec((BR, do2), lambda i: (i, 0)),
          pl.BlockSpec((BR, do2), lambda i: (i, 0)),
          pl.BlockSpec((BR, 1), lambda i: (i, 0)),
          pl.BlockSpec((BR, 1), lambda i: (i, 0)),
      ],
      out_shape=[
          jax.ShapeDtypeStruct((N, do2), jnp.float32),
          jax.ShapeDtypeStruct((N, do2), jnp.float32),
          jax.ShapeDtypeStruct((N, 1), jnp.float32),
          jax.ShapeDtypeStruct((N, 1), jnp.float32),
      ],
  )(num1, den1t, b1.reshape(1, din), W2,
    a_src2.reshape(dout, 1), a_dst2.reshape(dout, 1))
  return h2lo, h2hi, asv, adv


def _post_kernel(num_ref, den_ref, b_ref, w1_ref, b1_ref, w2_ref, b2_ref,
                 w11_ref, b11_ref, wv_ref, vb_ref, out_ref, vsum_ref):
  den = jnp.sum(den_ref[...], axis=1) + 1e-16
  inv = 1.0 / den[:, None]
  d2 = num_ref.shape[2]
  hg_lo = jnp.maximum(num_ref[0] * inv + b_ref[:, :d2], 0.0)
  hg_hi = jnp.maximum(num_ref[1] * inv + b_ref[:, d2:], 0.0)
  h = (jnp.dot(hg_lo, w1_ref[:d2], preferred_element_type=jnp.float32) +
       jnp.dot(hg_hi, w1_ref[d2:], preferred_element_type=jnp.float32) +
       b1_ref[...])
  h = jnp.dot(h, w2_ref[...], preferred_element_type=jnp.float32) + b2_ref[...]
  out_ref[...] = jnp.tanh(
      jnp.dot(h, w11_ref[...], preferred_element_type=jnp.float32) + b11_ref[...])
  vpart = jnp.sum(jnp.dot(h, wv_ref[...], preferred_element_type=jnp.float32))

  @pl.when(pl.program_id(0) == 0)
  def _():
    vsum_ref[...] = jnp.zeros_like(vsum_ref)

  vsum_ref[...] += jnp.reshape(vpart, (1, 1))

  @pl.when(pl.program_id(0) == ROWBLK - 1)
  def _():
    vsum_ref[...] = vsum_ref[...] / jnp.float32(N) + vb_ref[...]


def _post(num2, den2, b2, lin1_W, lin1_b, lin2_W, lin2_b, lin11_W, lin11_b,
          linV_W, linV_b):
  out, vsum = pl.pallas_call(
      _post_kernel,
      grid=(ROWBLK,),
      in_specs=[
          pl.BlockSpec((NC, BR, 64), lambda i: (0, i, 0)),
          pl.BlockSpec((BR, NS), lambda i: (i, 0)),
          pl.BlockSpec((1, 128), lambda i: (0, 0)),
          pl.BlockSpec((128, 64), lambda i: (0, 0)),
          pl.BlockSpec((1, 64), lambda i: (0, 0)),
          pl.BlockSpec((64, 64), lambda i: (0, 0)),
          pl.BlockSpec((1, 64), lambda i: (0, 0)),
          pl.BlockSpec((64, 64), lambda i: (0, 0)),
          pl.BlockSpec((1, 64), lambda i: (0, 0)),
          pl.BlockSpec((64, 1), lambda i: (0, 0)),
          pl.BlockSpec((1, 1), lambda i: (0, 0)),
      ],
      out_specs=[
          pl.BlockSpec((BR, 64), lambda i: (i, 0)),
          pl.BlockSpec((1, 1), lambda i: (0, 0)),
      ],
      out_shape=[
          jax.ShapeDtypeStruct((N, 64), jnp.float32),
          jax.ShapeDtypeStruct((1, 1), jnp.float32),
      ],
  )(num2, den2, b2.reshape(1, 128), lin1_W, lin1_b.reshape(1, 64),
    lin2_W, lin2_b.reshape(1, 64), lin11_W, lin11_b.reshape(1, 64), linV_W,
    linV_b.reshape(1, 1))
  return out, vsum


def kernel(x, edge_index, W1, a_src1, a_dst1, b1, W2, a_src2, a_dst2, b2,
           lin1_W, lin1_b, lin2_W, lin2_b, lin11_W, lin11_b, linV_W, linV_b):
  edges = edge_index.reshape(2, NS, NCHUNK2, CH)

  h1lo, h1hi, as1, ad1 = _proj(x, W1, a_src1, a_dst1)
  num1, den1 = _sc_gat64(h1lo, h1hi, as1.reshape(N), ad1.reshape(N), edges)
  h2lo, h2hi, as2, ad2 = _mid(num1, den1.T, b1, W2, a_src2, a_dst2)
  num2, den2 = _sc_gat128(h2lo, h2hi, as2.reshape(N), ad2.reshape(N), edges)
  out, value_arr = _post(num2, den2.T, b2, lin1_W, lin1_b, lin2_W, lin2_b,
                         lin11_W, lin11_b, linV_W, linV_b)
  return out, value_arr[0, 0]
